# speedup vs baseline: 2609.5156x; 2609.5156x over previous
import jax
import jax.numpy as jnp
from jax.experimental import pallas as pl
from jax.experimental.pallas import tpu as pltpu

_N = 2000
_D = 256


def _leaky(x, slope=0.2):
    return jnp.where(x >= 0, x, slope * x)


def _gat_body(scalars_ref, adjT_ref, x_ref, W0_ref, a0_ref, b0_ref,
              W1_ref, a1_ref, b1_ref, Wg_ref, bg_ref, wvec_ref,
              Wt_ref, bt_ref, Wc_ref, bc_ref, bag_ref, probs_ref):
    n = _N
    thr = scalars_ref[0]
    lnum = scalars_ref[1]

    adjT = adjT_ref[...]
    d_idx = jax.lax.broadcasted_iota(jnp.int32, (n, n), 0)
    s_idx = jax.lax.broadcasted_iota(jnp.int32, (n, n), 1)
    diag = d_idx == s_idx
    valid = jnp.logical_or(jnp.logical_and(adjT >= thr, jnp.logical_not(diag)),
                           diag)
    neg_inf = jnp.float32(-jnp.inf)

    def gat(x, W, a2, b_row):
        h = jnp.dot(x, W, preferred_element_type=jnp.float32)
        sv = jnp.dot(h, a2, preferred_element_type=jnp.float32)
        ss_col = sv[:, 0:1]
        sd_col = sv[:, 1:2]
        ss_row = jnp.transpose(ss_col)
        e = _leaky(sd_col + ss_row)
        e = jnp.where(valid, e, neg_inf)
        rmax = jnp.max(e, axis=1, keepdims=True)
        p = jnp.exp(e - rmax)
        p = jnp.where(valid, p, 0.0)
        den = jnp.sum(p, axis=1, keepdims=True) + 1e-16
        alpha = p / den
        out = jnp.dot(alpha, h, preferred_element_type=jnp.float32)
        return out + b_row

    x0 = x_ref[...]
    x1 = _leaky(gat(x0, W0_ref[...], a0_ref[...], b0_ref[...]))
    x1 = jnp.where(lnum > 0, x1, x0)
    x2 = _leaky(gat(x1, W1_ref[...], a1_ref[...], b1_ref[...]) + x0)
    x2 = jnp.where(lnum > 1, x2, x1)

    c = jnp.tanh(jnp.dot(x2, Wg_ref[...], preferred_element_type=jnp.float32)
                 + bg_ref[...])
    coeff = jnp.dot(c, wvec_ref[...], preferred_element_type=jnp.float32)
    m = jnp.max(coeff, axis=0, keepdims=True)
    w = jnp.exp(coeff - m)
    w = w / jnp.sum(w, axis=0, keepdims=True)
    bag = jnp.sum(w * x2, axis=0, keepdims=True)
    bag_ref[...] = bag

    bag_h = _leaky(jnp.dot(bag, Wt_ref[...], preferred_element_type=jnp.float32)
                   + bt_ref[...])
    probs_ref[...] = _leaky(
        jnp.dot(bag_h, Wc_ref[...], preferred_element_type=jnp.float32)
        + bc_ref[...])


def kernel(ins_feats, ins_adj, threshold, layer_num, W0, a_src0, a_dst0, b0,
           W1, a_src1, a_dst1, b1, Wg, bg, wvec, Wt, bt, Wc, bc):
    n, d = ins_feats.shape
    adjT = ins_adj.T
    scalars = jnp.stack([jnp.asarray(threshold, jnp.float32),
                         jnp.asarray(layer_num, jnp.float32)])
    a0 = jnp.stack([a_src0, a_dst0], axis=1)
    a1 = jnp.stack([a_src1, a_dst1], axis=1)

    smem = pl.BlockSpec(memory_space=pltpu.SMEM)
    vmem = pl.BlockSpec(memory_space=pltpu.VMEM)
    bag, probs = pl.pallas_call(
        _gat_body,
        out_shape=(jax.ShapeDtypeStruct((1, d), jnp.float32),
                   jax.ShapeDtypeStruct((1, 1), jnp.float32)),
        in_specs=[smem] + [vmem] * 15,
        out_specs=(vmem, vmem),
        compiler_params=pltpu.CompilerParams(
            vmem_limit_bytes=128 * 1024 * 1024),
    )(scalars, adjT, ins_feats, W0, a0, b0.reshape(1, d),
      W1, a1, b1.reshape(1, d), Wg, bg.reshape(1, d), wvec,
      Wt, bt.reshape(1, d // 2), Wc, bc.reshape(1, 1))
    return (probs.reshape(1), bag.reshape(d))


# speedup vs baseline: 3532.2029x; 1.3536x over previous
import jax
import jax.numpy as jnp
from jax.experimental import pallas as pl
from jax.experimental.pallas import tpu as pltpu

_N = 2000
_D = 256


def _leaky(x, slope=0.2):
    return jnp.where(x >= 0, x, slope * x)


def _gat_body(scalars_ref, adjT_ref, x_ref, W0_ref, a0_ref, b0_ref,
              W1_ref, a1_ref, b1_ref, Wg_ref, bg_ref, wvec_ref,
              Wt_ref, bt_ref, Wc_ref, bc_ref, bag_ref, probs_ref):
    n = _N
    thr = scalars_ref[0]
    lnum = scalars_ref[1]

    adj = adjT_ref[...]
    s_idx = jax.lax.broadcasted_iota(jnp.int32, (n, n), 0)
    d_idx = jax.lax.broadcasted_iota(jnp.int32, (n, n), 1)
    diag = s_idx == d_idx
    valid = jnp.logical_or(jnp.logical_and(adj >= thr, jnp.logical_not(diag)),
                           diag)
    neg_inf = jnp.float32(-jnp.inf)

    def gat(x, W, a2, b_row):
        h = jnp.dot(x, W, preferred_element_type=jnp.float32)
        sv = jnp.dot(h, a2, preferred_element_type=jnp.float32)
        ss_col = sv[:, 0:1]
        sd_row = jnp.transpose(sv[:, 1:2])
        e = _leaky(ss_col + sd_row)
        e = jnp.where(valid, e, neg_inf)
        cmax = jnp.max(e, axis=0, keepdims=True)
        p = jnp.exp(e - cmax)
        p = jnp.where(valid, p, 0.0)
        den = jnp.sum(p, axis=0, keepdims=True) + 1e-16
        alpha = p / den
        out = jax.lax.dot_general(alpha, h, (((0,), (0,)), ((), ())),
                                  precision=jax.lax.Precision.HIGHEST,
                                  preferred_element_type=jnp.float32)
        return out + b_row

    x0 = x_ref[...]
    x1 = _leaky(gat(x0, W0_ref[...], a0_ref[...], b0_ref[...]))
    x1 = jnp.where(lnum > 0, x1, x0)
    x2 = _leaky(gat(x1, W1_ref[...], a1_ref[...], b1_ref[...]) + x0)
    x2 = jnp.where(lnum > 1, x2, x1)

    c = jnp.tanh(jnp.dot(x2, Wg_ref[...], preferred_element_type=jnp.float32)
                 + bg_ref[...])
    coeff = jnp.dot(c, wvec_ref[...], preferred_element_type=jnp.float32)
    m = jnp.max(coeff, axis=0, keepdims=True)
    w = jnp.exp(coeff - m)
    w = w / jnp.sum(w, axis=0, keepdims=True)
    bag = jnp.sum(w * x2, axis=0, keepdims=True)
    bag_ref[...] = bag

    bag_h = _leaky(jnp.dot(bag, Wt_ref[...], preferred_element_type=jnp.float32)
                   + bt_ref[...])
    probs_ref[...] = _leaky(
        jnp.dot(bag_h, Wc_ref[...], preferred_element_type=jnp.float32,
                precision=jax.lax.Precision.HIGHEST)
        + bc_ref[...])


def kernel(ins_feats, ins_adj, threshold, layer_num, W0, a_src0, a_dst0, b0,
           W1, a_src1, a_dst1, b1, Wg, bg, wvec, Wt, bt, Wc, bc):
    n, d = ins_feats.shape
    scalars = jnp.stack([jnp.asarray(threshold, jnp.float32),
                         jnp.asarray(layer_num, jnp.float32)])
    a0 = jnp.stack([a_src0, a_dst0], axis=1)
    a1 = jnp.stack([a_src1, a_dst1], axis=1)

    smem = pl.BlockSpec(memory_space=pltpu.SMEM)
    vmem = pl.BlockSpec(memory_space=pltpu.VMEM)
    bag, probs = pl.pallas_call(
        _gat_body,
        out_shape=(jax.ShapeDtypeStruct((1, d), jnp.float32),
                   jax.ShapeDtypeStruct((1, 1), jnp.float32)),
        in_specs=[smem] + [vmem] * 15,
        out_specs=(vmem, vmem),
        compiler_params=pltpu.CompilerParams(
            vmem_limit_bytes=128 * 1024 * 1024),
    )(scalars, ins_adj, ins_feats, W0, a0, b0.reshape(1, d),
      W1, a1, b1.reshape(1, d), Wg, bg.reshape(1, d), wvec,
      Wt, bt.reshape(1, d // 2), Wc, bc.reshape(1, 1))
    return (probs.reshape(1), bag.reshape(d))


# speedup vs baseline: 3637.2836x; 1.0297x over previous
import jax
import jax.numpy as jnp
from jax.experimental import pallas as pl
from jax.experimental.pallas import tpu as pltpu

_N = 2000
_D = 256


def _leaky(x, slope=0.2):
    return jnp.where(x >= 0, x, slope * x)


def _gat_body(scalars_ref, adjT_ref, x_ref, W0_ref, a0_ref, b0_ref,
              W1_ref, a1_ref, b1_ref, Wg_ref, bg_ref, wvec_ref,
              Wt_ref, bt_ref, Wc_ref, bc_ref, bag_ref, probs_ref):
    n = _N
    thr = scalars_ref[0]
    lnum = scalars_ref[1]

    adj = adjT_ref[...]
    s_idx = jax.lax.broadcasted_iota(jnp.int32, (n, n), 0)
    d_idx = jax.lax.broadcasted_iota(jnp.int32, (n, n), 1)
    diag = s_idx == d_idx
    valid = jnp.logical_or(jnp.logical_and(adj >= thr, jnp.logical_not(diag)),
                           diag)
    neg_inf = jnp.float32(-jnp.inf)

    def gat(x, W, a2, b_row):
        h = jnp.dot(x, W, preferred_element_type=jnp.float32)
        sv = jnp.dot(h, a2, preferred_element_type=jnp.float32)
        ss_col = sv[:, 0:1]
        sd_row = jnp.transpose(sv[:, 1:2])
        e = _leaky(ss_col + sd_row)
        e = jnp.where(valid, e, neg_inf)
        cmax = jnp.max(e, axis=0, keepdims=True)
        p = jnp.exp(e - cmax)
        den = jnp.sum(p, axis=0, keepdims=True) + 1e-16
        alpha = p / den
        out = jax.lax.dot_general(alpha, h, (((0,), (0,)), ((), ())),
                                  precision=jax.lax.Precision.HIGHEST,
                                  preferred_element_type=jnp.float32)
        return out + b_row

    x0 = x_ref[...]
    x1 = _leaky(gat(x0, W0_ref[...], a0_ref[...], b0_ref[...]))
    x1 = jnp.where(lnum > 0, x1, x0)
    x2 = _leaky(gat(x1, W1_ref[...], a1_ref[...], b1_ref[...]) + x0)
    x2 = jnp.where(lnum > 1, x2, x1)

    c = jnp.tanh(jnp.dot(x2, Wg_ref[...], preferred_element_type=jnp.float32)
                 + bg_ref[...])
    coeff = jnp.dot(c, wvec_ref[...], preferred_element_type=jnp.float32)
    m = jnp.max(coeff, axis=0, keepdims=True)
    w = jnp.exp(coeff - m)
    w = w / jnp.sum(w, axis=0, keepdims=True)
    bag = jnp.sum(w * x2, axis=0, keepdims=True)
    bag_ref[...] = bag

    bag_h = _leaky(jnp.dot(bag, Wt_ref[...], preferred_element_type=jnp.float32)
                   + bt_ref[...])
    probs_ref[...] = _leaky(
        jnp.dot(bag_h, Wc_ref[...], preferred_element_type=jnp.float32,
                precision=jax.lax.Precision.HIGHEST)
        + bc_ref[...])


def kernel(ins_feats, ins_adj, threshold, layer_num, W0, a_src0, a_dst0, b0,
           W1, a_src1, a_dst1, b1, Wg, bg, wvec, Wt, bt, Wc, bc):
    n, d = ins_feats.shape
    scalars = jnp.stack([jnp.asarray(threshold, jnp.float32),
                         jnp.asarray(layer_num, jnp.float32)])
    a0 = jnp.stack([a_src0, a_dst0], axis=1)
    a1 = jnp.stack([a_src1, a_dst1], axis=1)

    smem = pl.BlockSpec(memory_space=pltpu.SMEM)
    vmem = pl.BlockSpec(memory_space=pltpu.VMEM)
    bag, probs = pl.pallas_call(
        _gat_body,
        out_shape=(jax.ShapeDtypeStruct((1, d), jnp.float32),
                   jax.ShapeDtypeStruct((1, 1), jnp.float32)),
        in_specs=[smem] + [vmem] * 15,
        out_specs=(vmem, vmem),
        compiler_params=pltpu.CompilerParams(
            vmem_limit_bytes=128 * 1024 * 1024),
    )(scalars, ins_adj, ins_feats, W0, a0, b0.reshape(1, d),
      W1, a1, b1.reshape(1, d), Wg, bg.reshape(1, d), wvec,
      Wt, bt.reshape(1, d // 2), Wc, bc.reshape(1, 1))
    return (probs.reshape(1), bag.reshape(d))


# speedup vs baseline: 3666.5912x; 1.0081x over previous
import jax
import jax.numpy as jnp
from jax.experimental import pallas as pl
from jax.experimental.pallas import tpu as pltpu

_N = 2000
_D = 256


def _leaky(x, slope=0.2):
    return jnp.maximum(x, slope * x)


def _gat_body(scalars_ref, adjT_ref, x_ref, W0_ref, a0_ref, b0_ref,
              W1_ref, a1_ref, b1_ref, Wg_ref, bg_ref, wvec_ref,
              Wt_ref, bt_ref, Wc_ref, bc_ref, bag_ref, probs_ref):
    n = _N
    thr = scalars_ref[0]
    lnum = scalars_ref[1]

    adj = adjT_ref[...]
    s_idx = jax.lax.broadcasted_iota(jnp.int32, (n, n), 0)
    d_idx = jax.lax.broadcasted_iota(jnp.int32, (n, n), 1)
    diag = s_idx == d_idx
    valid = jnp.logical_or(jnp.logical_and(adj >= thr, jnp.logical_not(diag)),
                           diag)
    neg_inf = jnp.float32(-jnp.inf)

    def gat(x, W, a2, b_row):
        h = jnp.dot(x, W, preferred_element_type=jnp.float32)
        sv = jnp.dot(h, a2, preferred_element_type=jnp.float32)
        ss_col = sv[:, 0:1]
        sd_row = jnp.transpose(sv[:, 1:2])
        e = _leaky(ss_col + sd_row)
        e = jnp.where(valid, e, neg_inf)
        cmax = jnp.max(e, axis=0, keepdims=True)
        p = jnp.exp(e - cmax)
        den = jnp.sum(p, axis=0, keepdims=True) + 1e-16
        alpha = p * (1.0 / den)
        out = jax.lax.dot_general(alpha, h, (((0,), (0,)), ((), ())),
                                  precision=jax.lax.Precision.HIGHEST,
                                  preferred_element_type=jnp.float32)
        return out + b_row

    x0 = x_ref[...]
    x1 = _leaky(gat(x0, W0_ref[...], a0_ref[...], b0_ref[...]))
    x1 = jnp.where(lnum > 0, x1, x0)
    x2 = _leaky(gat(x1, W1_ref[...], a1_ref[...], b1_ref[...]) + x0)
    x2 = jnp.where(lnum > 1, x2, x1)

    c = jnp.tanh(jnp.dot(x2, Wg_ref[...], preferred_element_type=jnp.float32)
                 + bg_ref[...])
    coeff = jnp.dot(c, wvec_ref[...], preferred_element_type=jnp.float32)
    m = jnp.max(coeff, axis=0, keepdims=True)
    w = jnp.exp(coeff - m)
    w = w / jnp.sum(w, axis=0, keepdims=True)
    bag = jnp.sum(w * x2, axis=0, keepdims=True)
    bag_ref[...] = bag

    bag_h = _leaky(jnp.dot(bag, Wt_ref[...], preferred_element_type=jnp.float32)
                   + bt_ref[...])
    probs_ref[...] = _leaky(
        jnp.dot(bag_h, Wc_ref[...], preferred_element_type=jnp.float32,
                precision=jax.lax.Precision.HIGHEST)
        + bc_ref[...])


def kernel(ins_feats, ins_adj, threshold, layer_num, W0, a_src0, a_dst0, b0,
           W1, a_src1, a_dst1, b1, Wg, bg, wvec, Wt, bt, Wc, bc):
    n, d = ins_feats.shape
    scalars = jnp.stack([jnp.asarray(threshold, jnp.float32),
                         jnp.asarray(layer_num, jnp.float32)])
    a0 = jnp.stack([a_src0, a_dst0], axis=1)
    a1 = jnp.stack([a_src1, a_dst1], axis=1)

    smem = pl.BlockSpec(memory_space=pltpu.SMEM)
    vmem = pl.BlockSpec(memory_space=pltpu.VMEM)
    bag, probs = pl.pallas_call(
        _gat_body,
        out_shape=(jax.ShapeDtypeStruct((1, d), jnp.float32),
                   jax.ShapeDtypeStruct((1, 1), jnp.float32)),
        in_specs=[smem] + [vmem] * 15,
        out_specs=(vmem, vmem),
        compiler_params=pltpu.CompilerParams(
            vmem_limit_bytes=128 * 1024 * 1024),
    )(scalars, ins_adj, ins_feats, W0, a0, b0.reshape(1, d),
      W1, a1, b1.reshape(1, d), Wg, bg.reshape(1, d), wvec,
      Wt, bt.reshape(1, d // 2), Wc, bc.reshape(1, 1))
    return (probs.reshape(1), bag.reshape(d))


# speedup vs baseline: 3671.4328x; 1.0013x over previous
import jax
import jax.numpy as jnp
from jax.experimental import pallas as pl
from jax.experimental.pallas import tpu as pltpu

_N = 2000
_D = 256


def _leaky(x, slope=0.2):
    return jnp.maximum(x, slope * x)


def _gat_body(scalars_ref, adj_ref, x_ref, W0_ref, a0_ref, b0_ref,
              W1_ref, a1_ref, b1_ref, Wg_ref, bg_ref, wvec_ref,
              Wt_ref, bt_ref, Wc_ref, bc_ref, bag_ref, probs_ref):
    n = _N
    thr = scalars_ref[0]
    lnum = scalars_ref[1]

    adj = adj_ref[...]
    s_idx = jax.lax.broadcasted_iota(jnp.int32, (n, n), 0)
    d_idx = jax.lax.broadcasted_iota(jnp.int32, (n, n), 1)
    diag = s_idx == d_idx
    valid = jnp.logical_or(jnp.logical_and(adj >= thr, jnp.logical_not(diag)),
                           diag)
    neg_inf = jnp.float32(-jnp.inf)

    def gat(x, W, a2, b_row):
        h = jnp.dot(x, W, preferred_element_type=jnp.float32)
        sv = jnp.dot(h, a2, preferred_element_type=jnp.float32)
        ss_col = sv[:, 0:1]
        sd_row = jnp.transpose(sv[:, 1:2])
        e = _leaky(ss_col + sd_row)
        e = jnp.where(valid, e, neg_inf)
        cmax = jnp.max(e, axis=0, keepdims=True)
        p = jnp.exp(e - cmax)
        den = jnp.sum(p, axis=0, keepdims=True) + 1e-16
        alpha = p * (1.0 / den)
        out = jax.lax.dot_general(alpha, h, (((0,), (0,)), ((), ())),
                                  precision=jax.lax.Precision.HIGHEST,
                                  preferred_element_type=jnp.float32)
        return out + b_row

    x0 = x_ref[...]
    x1 = _leaky(gat(x0, W0_ref[...], a0_ref[...], b0_ref[...]))
    x1 = jnp.where(lnum > 0, x1, x0)
    x2 = _leaky(gat(x1, W1_ref[...], a1_ref[...], b1_ref[...]) + x0)
    x2 = jnp.where(lnum > 1, x2, x1)

    c = jnp.tanh(jnp.dot(x2, Wg_ref[...], preferred_element_type=jnp.float32)
                 + bg_ref[...])
    coeff = jnp.dot(c, wvec_ref[...], preferred_element_type=jnp.float32)
    m = jnp.max(coeff, axis=0, keepdims=True)
    w = jnp.exp(coeff - m)
    w = w / jnp.sum(w, axis=0, keepdims=True)
    bag = jnp.sum(w * x2, axis=0, keepdims=True)
    bag_ref[...] = bag

    bag_h = _leaky(jnp.dot(bag, Wt_ref[...], preferred_element_type=jnp.float32)
                   + bt_ref[...])
    probs_ref[...] = _leaky(
        jnp.dot(bag_h, Wc_ref[...], preferred_element_type=jnp.float32,
                precision=jax.lax.Precision.HIGHEST)
        + bc_ref[...])


def kernel(ins_feats, ins_adj, threshold, layer_num, W0, a_src0, a_dst0, b0,
           W1, a_src1, a_dst1, b1, Wg, bg, wvec, Wt, bt, Wc, bc):
    n, d = ins_feats.shape
    scalars = jnp.stack([jnp.asarray(threshold, jnp.float32),
                         jnp.asarray(layer_num, jnp.float32)])
    a0 = jnp.stack([a_src0, a_dst0], axis=1)
    a1 = jnp.stack([a_src1, a_dst1], axis=1)

    smem = pl.BlockSpec(memory_space=pltpu.SMEM)
    vmem = pl.BlockSpec(memory_space=pltpu.VMEM)
    bag, probs = pl.pallas_call(
        _gat_body,
        out_shape=(jax.ShapeDtypeStruct((1, d), jnp.float32),
                   jax.ShapeDtypeStruct((1, 1), jnp.float32)),
        in_specs=[smem] + [vmem] * 15,
        out_specs=(vmem, vmem),
        compiler_params=pltpu.CompilerParams(
            vmem_limit_bytes=128 * 1024 * 1024),
    )(scalars, ins_adj, ins_feats, W0, a0, b0.reshape(1, d),
      W1, a1, b1.reshape(1, d), Wg, bg.reshape(1, d), wvec,
      Wt, bt.reshape(1, d // 2), Wc, bc.reshape(1, 1))
    return (probs.reshape(1), bag.reshape(d))
